# baseline (speedup 1.0000x reference)
import numpy as np
import jax
import jax.numpy as jnp
from jax.experimental import pallas as pl
from jax.experimental.pallas import tpu as pltpu

B = 128
V = 100000
BC = 7680
NB = (V + BC - 1) // BC

_TINY = np.float32(np.finfo(np.float32).tiny)

_KS0 = np.uint32(0)
_KS1 = np.uint32(42)
_KS2 = np.uint32(_KS0 ^ _KS1 ^ np.uint32(0x1BD11BDA))

_ROT_A = (13, 15, 26, 6)
_ROT_B = (17, 29, 16, 24)


def _rotl(x, d):
    return (x << np.uint32(d)) | (x >> np.uint32(32 - d))


def _threefry_bits(x0, x1):
    x0 = x0 + _KS0
    x1 = x1 + _KS1
    inj = (
        (_KS1, _KS2 + np.uint32(1)),
        (_KS2, _KS0 + np.uint32(2)),
        (_KS0, _KS1 + np.uint32(3)),
        (_KS1, _KS2 + np.uint32(4)),
        (_KS2, _KS0 + np.uint32(5)),
    )
    for g in range(5):
        rots = _ROT_A if g % 2 == 0 else _ROT_B
        for r in rots:
            x0 = x0 + x1
            x1 = _rotl(x1, r)
            x1 = x0 ^ x1
        a, b = inj[g]
        x0 = x0 + a
        x1 = x1 + b
    return x0 ^ x1


def _gumbel_vals(tile, col):
    row = jax.lax.broadcasted_iota(jnp.uint32, tile.shape, 0)
    x1 = row * np.uint32(V) + col
    bits = _threefry_bits(jnp.zeros_like(x1), x1)

    fb = (bits >> np.uint32(9)) | np.uint32(0x3F800000)
    flo = pltpu.bitcast(fb, jnp.float32) - np.float32(1.0)
    u = jnp.maximum(flo, _TINY)
    return -jnp.log(-jnp.log(u)) + tile


def _sample_kernel(logits_ref, out_ref, bestv_ref, besti_ref):
    j = pl.program_id(0)
    col = (jax.lax.broadcasted_iota(jnp.uint32, (B, BC), 1)
           + (j * BC).astype(jnp.uint32))
    vals = _gumbel_vals(logits_ref[...], col)
    vals = jnp.where(col < np.uint32(V), vals, -jnp.inf)

    m = jnp.max(vals, axis=1, keepdims=True)
    idx = jnp.min(
        jnp.where(vals == m, col.astype(jnp.int32),
                  jnp.int32(np.iinfo(np.int32).max)),
        axis=1, keepdims=True)

    @pl.when(j == 0)
    def _():
        bestv_ref[...] = m
        besti_ref[...] = idx

    @pl.when(j > 0)
    def _():
        upd = m > bestv_ref[...]
        besti_ref[...] = jnp.where(upd, idx, besti_ref[...])
        bestv_ref[...] = jnp.where(upd, m, bestv_ref[...])

    @pl.when(j == NB - 1)
    def _():
        out_ref[...] = besti_ref[...]


@jax.jit
def kernel(logits):
    out = pl.pallas_call(
        _sample_kernel,
        grid=(NB,),
        in_specs=[pl.BlockSpec((B, BC), lambda j: (0, j))],
        out_specs=pl.BlockSpec((B, 1), lambda j: (0, 0)),
        out_shape=jax.ShapeDtypeStruct((B, 1), jnp.int32),
        scratch_shapes=[
            pltpu.VMEM((B, 1), jnp.float32),
            pltpu.VMEM((B, 1), jnp.int32),
        ],
    )(logits)
    return out[:, 0].astype(jnp.int64)


# speedup vs baseline: 1.7605x; 1.7605x over previous
import numpy as np
import jax
import jax.numpy as jnp
from jax.experimental import pallas as pl
from jax.experimental.pallas import tpu as pltpu

B = 128
V = 100000
BC = 2048
NB = (V + BC - 1) // BC
NBUF = 3
_VPAD = -(-V // 128) * 128
_LAST = _VPAD - BC

_TINY = np.float32(np.finfo(np.float32).tiny)

_KS0 = np.uint32(0)
_KS1 = np.uint32(42)
_KS2 = np.uint32(_KS0 ^ _KS1 ^ np.uint32(0x1BD11BDA))

_ROT_A = (13, 15, 26, 6)
_ROT_B = (17, 29, 16, 24)


def _rotl(x, d):
    return (x << np.uint32(d)) | (x >> np.uint32(32 - d))


def _threefry_bits(x0, x1):
    x0 = x0 + _KS0
    x1 = x1 + _KS1
    inj = (
        (_KS1, _KS2 + np.uint32(1)),
        (_KS2, _KS0 + np.uint32(2)),
        (_KS0, _KS1 + np.uint32(3)),
        (_KS1, _KS2 + np.uint32(4)),
        (_KS2, _KS0 + np.uint32(5)),
    )
    for g in range(5):
        rots = _ROT_A if g % 2 == 0 else _ROT_B
        for r in rots:
            x0 = x0 + x1
            x1 = _rotl(x1, r)
            x1 = x0 ^ x1
        a, b = inj[g]
        x0 = x0 + a
        x1 = x1 + b
    return x0 ^ x1


def _gumbel_vals(tile, col):
    row = jax.lax.broadcasted_iota(jnp.uint32, tile.shape, 0)
    x1 = row * np.uint32(V) + col
    bits = _threefry_bits(jnp.zeros_like(x1), x1)

    fb = (bits >> np.uint32(9)) | np.uint32(0x3F800000)
    flo = pltpu.bitcast(fb, jnp.float32) - np.float32(1.0)
    u = jnp.maximum(flo, _TINY)
    return -jnp.log(-jnp.log(u)) + tile


def _sample_kernel(logits_hbm, out_ref, buf, bestv_ref, besti_ref, sems):
    j = pl.program_id(0)

    def _copy(t, slot):
        st = jnp.minimum(t * BC, _LAST)
        return pltpu.make_async_copy(
            logits_hbm.at[:, pl.ds(st, BC)], buf.at[slot],
            sems.at[slot])

    @pl.when(j == 0)
    def _():
        for t in range(NBUF):
            _copy(t, t).start()

    @pl.when(jnp.logical_and(j > 0, j + NBUF - 1 < NB))
    def _():
        _copy(j + NBUF - 1, (j + NBUF - 1) % NBUF).start()

    slot = j % NBUF
    _copy(j, slot).wait()

    col = (jax.lax.broadcasted_iota(jnp.uint32, (B, BC), 1)
           + jnp.minimum(j * BC, _LAST).astype(jnp.uint32))
    vals = _gumbel_vals(buf[slot], col)
    vals = jnp.where(col < np.uint32(V), vals, -jnp.inf)

    m = jnp.max(vals, axis=1, keepdims=True)
    idx = jnp.min(
        jnp.where(vals == m, col.astype(jnp.int32),
                  jnp.int32(np.iinfo(np.int32).max)),
        axis=1, keepdims=True)

    @pl.when(j == 0)
    def _():
        bestv_ref[...] = m
        besti_ref[...] = idx

    @pl.when(j > 0)
    def _():
        upd = m > bestv_ref[...]
        besti_ref[...] = jnp.where(upd, idx, besti_ref[...])
        bestv_ref[...] = jnp.where(upd, m, bestv_ref[...])

    @pl.when(j == NB - 1)
    def _():
        out_ref[...] = besti_ref[...]


@jax.jit
def kernel(logits):
    out = pl.pallas_call(
        _sample_kernel,
        grid=(NB,),
        in_specs=[pl.BlockSpec(memory_space=pl.ANY)],
        out_specs=pl.BlockSpec((B, 1), lambda j: (0, 0)),
        out_shape=jax.ShapeDtypeStruct((B, 1), jnp.int32),
        scratch_shapes=[
            pltpu.VMEM((NBUF, B, BC), jnp.float32),
            pltpu.VMEM((B, 1), jnp.float32),
            pltpu.VMEM((B, 1), jnp.int32),
            pltpu.SemaphoreType.DMA((NBUF,)),
        ],
    )(logits)
    return out[:, 0].astype(jnp.int64)
